# baseline (speedup 1.0000x reference)
import functools

import jax
import jax.numpy as jnp
from jax import lax
from jax.experimental import pallas as pl
from jax.experimental.pallas import tpu as pltpu
from jax.experimental.pallas import tpu_sc as plsc

EMBED_DIM = 64
NC = 2
NS = 16
NW = NC * NS
CHUNK = 128
HALF = CHUNK // 2
NBUF = 8


@functools.lru_cache(maxsize=None)
def _build_gather(total_rows):
    rows_per_w = total_rows // NW
    n_chunks = rows_per_w // CHUNK
    out_rows_per_w = rows_per_w // 2
    mesh = plsc.VectorSubcoreMesh(core_axis_name="c", subcore_axis_name="s")

    @functools.partial(
        pl.kernel,
        out_type=jax.ShapeDtypeStruct((total_rows // 2, 2 * EMBED_DIM),
                                      jnp.float32),
        mesh=mesh,
        compiler_params=pltpu.CompilerParams(use_tc_tiling_on_sc=False),
        scratch_types=[
            pltpu.VMEM((n_chunks, CHUNK), jnp.int32),
        ] + [pltpu.VMEM((CHUNK, EMBED_DIM), jnp.float32)] * NBUF
          + [pltpu.SemaphoreType.DMA] * (2 * NBUF),
    )
    def gather_kernel(idx_hbm, table_hbm, out_hbm, idx_v, *rest):
        bufs = rest[:NBUF]
        gsems = rest[NBUF:2 * NBUF]
        ssems = rest[2 * NBUF:]
        wid = lax.axis_index("s") * NC + lax.axis_index("c")
        base = wid * out_rows_per_w
        pltpu.sync_copy(idx_hbm.at[pl.ds(wid * n_chunks, n_chunks)], idx_v)

        def start_gathers(chunk, b):
            h0 = pltpu.async_copy(
                table_hbm.at[idx_v.at[chunk, pl.ds(0, HALF)]],
                bufs[b].at[pl.ds(0, HALF)], gsems[b])
            h1 = pltpu.async_copy(
                table_hbm.at[idx_v.at[chunk, pl.ds(HALF, HALF)]],
                bufs[b].at[pl.ds(HALF, HALF)], gsems[b])
            return (h0, h1)

        def start_scatters(chunk, b):
            o = base + chunk * HALF
            h0 = pltpu.async_copy(
                bufs[b].at[pl.ds(0, HALF)],
                out_hbm.at[pl.ds(o, HALF), pl.ds(0, EMBED_DIM)], ssems[b])
            h1 = pltpu.async_copy(
                bufs[b].at[pl.ds(HALF, HALF)],
                out_hbm.at[pl.ds(o, HALF), pl.ds(EMBED_DIM, EMBED_DIM)],
                ssems[b])
            return (h0, h1)

        gathers = [None] * NBUF
        scatters = [None] * NBUF
        for i in range(min(NBUF - 1, n_chunks)):
            gathers[i] = start_gathers(i, i)
        for i in range(n_chunks):
            b = i % NBUF
            j = i + NBUF - 1
            if j < n_chunks:
                jb = j % NBUF
                if scatters[jb] is not None:
                    scatters[jb][0].wait()
                    scatters[jb][1].wait()
                    scatters[jb] = None
                gathers[jb] = start_gathers(j, jb)
            gathers[b][0].wait()
            gathers[b][1].wait()
            scatters[b] = start_scatters(i, b)
        for s in scatters:
            if s is not None:
                s[0].wait()
                s[1].wait()

    return gather_kernel


def kernel(c, img, q, cap_len, table):
    batch, cap_len_dim = c.shape
    total_rows = batch * cap_len_dim
    n_total_chunks = total_rows // CHUNK
    c_pairs = c.reshape(n_total_chunks, HALF, 2).astype(jnp.int32)
    idx = jnp.concatenate([c_pairs[:, :, 0], c_pairs[:, :, 1]], axis=1)
    flat = _build_gather(total_rows)(idx, table)
    c_emb = flat.reshape(batch, cap_len_dim, EMBED_DIM)
    return (img, c_emb, c, cap_len)


# speedup vs baseline: 1.5876x; 1.5876x over previous
import functools

import jax
import jax.numpy as jnp
from jax import lax
from jax.experimental import pallas as pl
from jax.experimental.pallas import tpu as pltpu
from jax.experimental.pallas import tpu_sc as plsc

EMBED_DIM = 64
NC = 2
NS = 16
NW = NC * NS
CHUNK = 128
NBUF = 8


@functools.lru_cache(maxsize=None)
def _build_gather(total_rows):
    rows_per_w = total_rows // NW
    n_chunks = rows_per_w // CHUNK
    mesh = plsc.VectorSubcoreMesh(core_axis_name="c", subcore_axis_name="s")

    @functools.partial(
        pl.kernel,
        out_type=jax.ShapeDtypeStruct((total_rows, EMBED_DIM), jnp.float32),
        mesh=mesh,
        compiler_params=pltpu.CompilerParams(use_tc_tiling_on_sc=False),
        scratch_types=[
            pltpu.VMEM((n_chunks, CHUNK), jnp.int32),
        ] + [pltpu.VMEM((CHUNK, EMBED_DIM), jnp.float32)] * NBUF
          + [pltpu.SemaphoreType.DMA] * (2 * NBUF),
    )
    def gather_kernel(idx_hbm, table_hbm, out_hbm, idx_v, *rest):
        bufs = rest[:NBUF]
        gsems = rest[NBUF:2 * NBUF]
        ssems = rest[2 * NBUF:]
        wid = lax.axis_index("s") * NC + lax.axis_index("c")
        base = wid * rows_per_w
        pltpu.sync_copy(idx_hbm.at[pl.ds(wid * n_chunks, n_chunks)], idx_v)

        gathers = [None] * NBUF
        scatters = [None] * NBUF
        for i in range(min(NBUF - 1, n_chunks)):
            gathers[i] = pltpu.async_copy(
                table_hbm.at[idx_v.at[i]], bufs[i], gsems[i])
        for i in range(n_chunks):
            b = i % NBUF
            j = i + NBUF - 1
            if j < n_chunks:
                jb = j % NBUF
                if scatters[jb] is not None:
                    scatters[jb].wait()
                    scatters[jb] = None
                gathers[jb] = pltpu.async_copy(
                    table_hbm.at[idx_v.at[j]], bufs[jb], gsems[jb])
            gathers[b].wait()
            scatters[b] = pltpu.async_copy(
                bufs[b], out_hbm.at[pl.ds(base + i * CHUNK, CHUNK)],
                ssems[b])
        for s in scatters:
            if s is not None:
                s.wait()

    return gather_kernel


def kernel(c, img, q, cap_len, table):
    batch, cap_len_dim = c.shape
    total_rows = batch * cap_len_dim
    idx = c.T.reshape(total_rows // CHUNK, CHUNK).astype(jnp.int32)
    flat = _build_gather(total_rows)(idx, table)
    c_emb = flat.reshape(cap_len_dim, batch, EMBED_DIM).transpose(1, 0, 2)
    return (img, c_emb, c, cap_len)
